# baseline (device time: 17569 ns/iter reference)
import jax
import jax.numpy as jnp
from jax import lax
from jax.experimental import pallas as pl
from jax.experimental.pallas import tpu as pltpu

N_DEV = 8
H = 2


def kernel(A, B):
    m, k = A.shape
    k2, n = B.shape
    mc = m // N_DEV
    hc = mc // H

    def body(a_ref, b_ref, out_ref, part_ref,
             p1h0, p1h1, gh0, gh1,
             sp1h0, sp1h1, rp1h0, rp1h1,
             sp2h0, sp2h1, rp2h0, rp2h1):
        my = lax.axis_index("i")
        p1_half = [p1h0, p1h1]
        g_half = [gh0, gh1]
        send_p1 = [sp1h0, sp1h1]
        recv_p1 = [rp1h0, rp1h1]
        send_p2 = [sp2h0, sp2h1]
        recv_p2 = [rp2h0, rp2h1]

        barrier_sem = pltpu.get_barrier_semaphore()
        for d in range(1, N_DEV):
            pl.semaphore_signal(
                barrier_sem, inc=1,
                device_id=((my + d) % N_DEV,),
                device_id_type=pl.DeviceIdType.MESH,
            )

        part_ref[:, :, :, :] = jnp.dot(
            a_ref[:, :].astype(jnp.bfloat16),
            b_ref[:, :].astype(jnp.bfloat16),
            preferred_element_type=jnp.float32,
        ).astype(jnp.bfloat16).reshape(N_DEV, H, hc, n)
        for h in range(H):
            p1_half[h][pl.ds(my, 1), :, :] = part_ref[pl.ds(my, 1), h]

        pl.semaphore_wait(barrier_sem, N_DEV - 1)

        p1_sends = []
        for h in range(H):
            for d in range(1, N_DEV):
                tgt = (my + d) % N_DEV
                rdma = pltpu.make_async_remote_copy(
                    src_ref=part_ref.at[tgt, h],
                    dst_ref=p1_half[h].at[my],
                    send_sem=send_p1[h].at[d - 1],
                    recv_sem=recv_p1[h].at[my],
                    device_id=(tgt,),
                    device_id_type=pl.DeviceIdType.MESH,
                )
                rdma.start()
                p1_sends.append(rdma)

        p2_sends = []
        for h in range(H):
            z = p1_half[h][pl.ds(my, 1), :, :].astype(jnp.float32)
            for d in range(1, N_DEV):
                src = (my + d) % N_DEV
                recv = pltpu.make_async_remote_copy(
                    src_ref=p1_half[h].at[src],
                    dst_ref=p1_half[h].at[src],
                    send_sem=send_p1[h].at[d - 1],
                    recv_sem=recv_p1[h].at[src],
                    device_id=(src,),
                    device_id_type=pl.DeviceIdType.MESH,
                )
                recv.wait_recv()
                z += p1_half[h][pl.ds(src, 1), :, :].astype(jnp.float32)
            z = z[0]
            silu = z / (1.0 + jnp.exp(-z))
            g_half[h][pl.ds(my, 1), :, :] = silu.astype(jnp.bfloat16)[None]
            out_ref[pl.ds(my * mc + h * hc, hc), :] = silu
            for d in range(1, N_DEV):
                tgt = (my + d) % N_DEV
                rdma = pltpu.make_async_remote_copy(
                    src_ref=g_half[h].at[my],
                    dst_ref=g_half[h].at[my],
                    send_sem=send_p2[h].at[d - 1],
                    recv_sem=recv_p2[h].at[my],
                    device_id=(tgt,),
                    device_id_type=pl.DeviceIdType.MESH,
                )
                rdma.start()
                p2_sends.append(rdma)

        for rdma in p1_sends:
            rdma.wait_send()

        for h in range(H):
            for d in range(1, N_DEV):
                src = (my + d) % N_DEV
                recv = pltpu.make_async_remote_copy(
                    src_ref=g_half[h].at[src],
                    dst_ref=g_half[h].at[src],
                    send_sem=send_p2[h].at[d - 1],
                    recv_sem=recv_p2[h].at[src],
                    device_id=(src,),
                    device_id_type=pl.DeviceIdType.MESH,
                )
                recv.wait_recv()
                out_ref[pl.ds(src * mc + h * hc, hc), :] = (
                    g_half[h][pl.ds(src, 1), :, :].astype(jnp.float32)[0]
                )

        for rdma in p2_sends:
            rdma.wait_send()

    hbuf = pltpu.VMEM((N_DEV, m // N_DEV // H, n), jnp.bfloat16)
    return pl.pallas_call(
        body,
        out_shape=jax.ShapeDtypeStruct((m, n), jnp.float32),
        in_specs=[
            pl.BlockSpec(memory_space=pltpu.VMEM),
            pl.BlockSpec(memory_space=pltpu.VMEM),
        ],
        out_specs=pl.BlockSpec(memory_space=pltpu.VMEM),
        scratch_shapes=[
            pltpu.VMEM((N_DEV, H, m // N_DEV // H, n), jnp.bfloat16),
            hbuf, hbuf,
            hbuf, hbuf,
            pltpu.SemaphoreType.DMA((N_DEV - 1,)),
            pltpu.SemaphoreType.DMA((N_DEV - 1,)),
            pltpu.SemaphoreType.DMA((N_DEV,)),
            pltpu.SemaphoreType.DMA((N_DEV,)),
            pltpu.SemaphoreType.DMA((N_DEV - 1,)),
            pltpu.SemaphoreType.DMA((N_DEV - 1,)),
            pltpu.SemaphoreType.DMA((N_DEV,)),
            pltpu.SemaphoreType.DMA((N_DEV,)),
        ],
        compiler_params=pltpu.CompilerParams(collective_id=0),
    )(A, B)
